# baseline (speedup 1.0000x reference)
import jax
import jax.numpy as jnp
from jax.experimental import pallas as pl
from jax.experimental.pallas import tpu as pltpu

D_MODEL = 512
K_NEAREST = 100
NQ = 3000
NKV = 16384
LN_EPS = 1e-5

BQ = 64
NQ_PAD = 3072
GRIDQ = NQ_PAD // BQ
SCALE = D_MODEL ** (-0.5)
INF_BITS = 0x7F800000

CK1 = 2048
NC1 = NKV // CK1
CK2 = 2048
NC2 = NKV // CK2
BQ2 = 512
GRIDQ2 = NQ_PAD // BQ2


def _dist_bits(qp_ref, kx, ky, kz):
    qx = qp_ref[:, 0:1]
    qy = qp_ref[:, 1:2]
    qz = qp_ref[:, 2:3]
    dx = qx - kx
    dy = qy - ky
    dz = qz - kz
    d2 = dx * dx + dy * dy + dz * dz
    return jax.lax.bitcast_convert_type(jnp.sqrt(d2), jnp.int32)


def _thresh_body(qp_ref, kpt_ref, kth_ref, cut_ref, bits_ref,
                 lo_ref, hi_ref, t_ref, fnd_ref, cl_ref, ch_ref):
    lo0 = jnp.full((BQ, 1), INF_BITS, jnp.int32)
    hi0 = jnp.zeros((BQ, 1), jnp.int32)
    for c in range(NC1):
        sl = pl.ds(c * CK1, CK1)
        b = _dist_bits(qp_ref, kpt_ref[0:1, sl],
                       kpt_ref[1:2, sl], kpt_ref[2:3, sl])
        bits_ref[:, sl] = b
        lo0 = jnp.minimum(lo0, jnp.min(b, axis=1, keepdims=True))
        hi0 = jnp.maximum(hi0, jnp.max(b, axis=1, keepdims=True))

    def count_le(mid):
        tot = jnp.zeros((BQ, 1), jnp.int32)
        for c in range(NC1):
            sl = pl.ds(c * CK1, CK1)
            tot = tot + jnp.sum((bits_ref[:, sl] <= mid).astype(jnp.int32),
                                axis=1, keepdims=True)
        return tot

    lo_ref[...] = jnp.broadcast_to(lo0 - 1, (BQ, 128))
    hi_ref[...] = jnp.broadcast_to(hi0, (BQ, 128))
    t_ref[...] = jnp.zeros((BQ, 128), jnp.int32)
    fnd_ref[...] = jnp.zeros((BQ, 128), jnp.int32)
    cl_ref[...] = jnp.zeros((BQ, 128), jnp.float32)
    ch_ref[...] = jnp.full((BQ, 128), float(NKV), jnp.float32)

    def bs_cond(carry):
        it, notdone = carry
        return jnp.logical_and(it < 40, notdone)

    def bs_step(carry):
        it, _ = carry
        lo = lo_ref[:, 0:1]
        hi = hi_ref[:, 0:1]
        found = fnd_ref[:, 0:1]
        cl = cl_ref[:, 0:1]
        ch = ch_ref[:, 0:1]

        lov = jax.lax.bitcast_convert_type(lo, jnp.float32)
        hiv = jax.lax.bitcast_convert_type(hi, jnp.float32)
        frac = (K_NEAREST - cl) / jnp.maximum(ch - cl, 1.0)
        frac = jnp.clip(frac, 0.002, 0.998)
        lo3 = lov * lov * lov
        hi3 = hiv * hiv * hiv
        t3 = jnp.maximum(lo3 + frac * (hi3 - lo3), 1e-30)
        tv = jnp.exp(jnp.log(t3) * (1.0 / 3.0))
        interp_mid = jax.lax.bitcast_convert_type(tv, jnp.int32)

        bisect_mid = lo + jax.lax.div(hi - lo, 2)
        mid = jnp.where(it < 16, interp_mid, bisect_mid)
        mid = jnp.clip(mid, lo + 1, hi - 1)

        cnt = count_le(mid)
        newly = (cnt == K_NEAREST) & (found == 0)
        t_ref[...] = jnp.broadcast_to(
            jnp.where(newly, mid, t_ref[:, 0:1]), (BQ, 128))
        found = found | newly.astype(jnp.int32)
        fnd_ref[...] = jnp.broadcast_to(found, (BQ, 128))
        take = cnt >= K_NEAREST
        lo_n = jnp.where(take, lo, mid)
        hi_n = jnp.where(take, mid, hi)
        lo_ref[...] = jnp.broadcast_to(lo_n, (BQ, 128))
        hi_ref[...] = jnp.broadcast_to(hi_n, (BQ, 128))
        cntf = cnt.astype(jnp.float32)
        cl_ref[...] = jnp.broadcast_to(jnp.where(take, cl, cntf), (BQ, 128))
        ch_ref[...] = jnp.broadcast_to(jnp.where(take, cntf, ch), (BQ, 128))
        resolved = (found == 1) | (hi_n - lo_n <= 1)
        return it + 1, jnp.logical_not(jnp.all(resolved))

    jax.lax.while_loop(bs_cond, bs_step, (jnp.int32(0), True))
    found = fnd_ref[:, 0:1]
    kth = jnp.where(found == 1, t_ref[:, 0:1], hi_ref[:, 0:1])
    tie = jnp.logical_not(jnp.all(found == 1))

    def tie_fix(_):
        def count_lt():
            tot = jnp.zeros((BQ, 1), jnp.int32)
            for c in range(NC1):
                sl = pl.ds(c * CK1, CK1)
                tot = tot + jnp.sum((bits_ref[:, sl] < kth).astype(jnp.int32),
                                    axis=1, keepdims=True)
            return tot

        need = K_NEAREST - count_lt()
        iot = jax.lax.broadcasted_iota(jnp.int32, (BQ, CK1), 1)

        def count_eq_upto(mid):
            tot = jnp.zeros((BQ, 1), jnp.int32)
            for c in range(NC1):
                sl = pl.ds(c * CK1, CK1)
                hit = (bits_ref[:, sl] == kth) & ((iot + c * CK1) <= mid)
                tot = tot + jnp.sum(hit.astype(jnp.int32), axis=1,
                                    keepdims=True)
            return tot

        def ix_step(_, lohi):
            lo, hi = lohi
            mid = lo + jax.lax.div(hi - lo, 2)
            take = count_eq_upto(mid) >= need
            return jnp.where(take, lo, mid + 1), jnp.where(take, mid, hi)

        lo = jnp.zeros((BQ, 1), jnp.int32)
        hi = jnp.full((BQ, 1), NKV - 1, jnp.int32)
        _, cut = jax.lax.fori_loop(0, 14, ix_step, (lo, hi))
        return cut

    cut = jax.lax.cond(tie, tie_fix,
                       lambda _: jnp.full((BQ, 1), NKV - 1, jnp.int32),
                       None)
    kth_ref[...] = jnp.broadcast_to(kth, (BQ, 128))
    cut_ref[...] = jnp.broadcast_to(cut, (BQ, 128))


def _attn_body(qf_ref, kf_ref, vf_ref, qp_ref, kpt_ref, kth_ref, cut_ref,
               g_ref, b_ref, out_ref, acc_ref, m_ref, l_ref):
    j = pl.program_id(1)

    bits = _dist_bits(qp_ref, kpt_ref[0:1, :], kpt_ref[1:2, :],
                      kpt_ref[2:3, :])
    kth = kth_ref[:, 0:1]
    cut = cut_ref[:, 0:1]
    iot = jax.lax.broadcasted_iota(jnp.int32, (BQ2, CK2), 1) + j * CK2
    mask = (bits < kth) | ((bits == kth) & (iot <= cut))

    s = jax.lax.dot_general(qf_ref[...], kf_ref[...],
                            (((1,), (1,)), ((), ())),
                            preferred_element_type=jnp.float32) * SCALE
    sm = jnp.where(mask, s, -jnp.inf)
    mc = jnp.max(sm, axis=1, keepdims=True)

    @pl.when(j == 0)
    def _init():
        m_ref[...] = jnp.broadcast_to(mc, (BQ2, 128))
        p = jnp.where(mask, jnp.exp(s - mc), 0.0)
        l_ref[...] = jnp.broadcast_to(
            jnp.sum(p, axis=1, keepdims=True), (BQ2, 128))
        acc_ref[...] = jax.lax.dot_general(
            p, vf_ref[...], (((1,), (0,)), ((), ())),
            preferred_element_type=jnp.float32)

    @pl.when(j > 0)
    def _update():
        m_old = m_ref[:, 0:1]
        m_new = jnp.maximum(m_old, mc)
        corr = jnp.where(m_old == -jnp.inf, 0.0, jnp.exp(m_old - m_new))
        p = jnp.where(mask, jnp.exp(s - m_new), 0.0)
        m_ref[...] = jnp.broadcast_to(m_new, (BQ2, 128))
        l_ref[...] = (l_ref[...] * corr +
                      jnp.broadcast_to(jnp.sum(p, axis=1, keepdims=True),
                                       (BQ2, 128)))
        acc_ref[...] = (acc_ref[...] * corr +
                        jax.lax.dot_general(
                            p, vf_ref[...], (((1,), (0,)), ((), ())),
                            preferred_element_type=jnp.float32))

    @pl.when(j == NC2 - 1)
    def _finalize():
        x = acc_ref[...] / l_ref[:, 0:1]
        t = x + x
        mu = jnp.mean(t, axis=1, keepdims=True)
        var = jnp.mean((t - mu) ** 2, axis=1, keepdims=True)
        xh = (t - mu) / jnp.sqrt(var + LN_EPS)
        out_ref[...] = xh * g_ref[0:1, :] + b_ref[0:1, :]


@jax.jit
def kernel(res_feat, q_feat, k_feat, v_feat, q_pos, k_pos, ln_gamma, ln_beta):
    del res_feat
    qf = jnp.pad(q_feat, ((0, NQ_PAD - NQ), (0, 0)))
    qp = jnp.pad(q_pos, ((0, NQ_PAD - NQ), (0, 125)))
    kpt = jnp.pad(k_pos.T, ((0, 5), (0, 0)))
    g2 = jnp.broadcast_to(ln_gamma[None, :], (8, D_MODEL))
    b2 = jnp.broadcast_to(ln_beta[None, :], (8, D_MODEL))

    kth, cut = pl.pallas_call(
        _thresh_body,
        grid=(GRIDQ,),
        in_specs=[
            pl.BlockSpec((BQ, 128), lambda i: (i, 0)),
            pl.BlockSpec((8, NKV), lambda i: (0, 0)),
        ],
        out_specs=[
            pl.BlockSpec((BQ, 128), lambda i: (i, 0)),
            pl.BlockSpec((BQ, 128), lambda i: (i, 0)),
        ],
        out_shape=[
            jax.ShapeDtypeStruct((NQ_PAD, 128), jnp.int32),
            jax.ShapeDtypeStruct((NQ_PAD, 128), jnp.int32),
        ],
        scratch_shapes=[pltpu.VMEM((BQ, NKV), jnp.int32),
                        pltpu.VMEM((BQ, 128), jnp.int32),
                        pltpu.VMEM((BQ, 128), jnp.int32),
                        pltpu.VMEM((BQ, 128), jnp.int32),
                        pltpu.VMEM((BQ, 128), jnp.int32),
                        pltpu.VMEM((BQ, 128), jnp.float32),
                        pltpu.VMEM((BQ, 128), jnp.float32)],
    )(qp, kpt)

    out = pl.pallas_call(
        _attn_body,
        grid=(GRIDQ2, NC2),
        in_specs=[
            pl.BlockSpec((BQ2, D_MODEL), lambda i, j: (i, 0)),
            pl.BlockSpec((CK2, D_MODEL), lambda i, j: (j, 0)),
            pl.BlockSpec((CK2, D_MODEL), lambda i, j: (j, 0)),
            pl.BlockSpec((BQ2, 128), lambda i, j: (i, 0)),
            pl.BlockSpec((8, CK2), lambda i, j: (0, j)),
            pl.BlockSpec((BQ2, 128), lambda i, j: (i, 0)),
            pl.BlockSpec((BQ2, 128), lambda i, j: (i, 0)),
            pl.BlockSpec((8, D_MODEL), lambda i, j: (0, 0)),
            pl.BlockSpec((8, D_MODEL), lambda i, j: (0, 0)),
        ],
        out_specs=pl.BlockSpec((BQ2, D_MODEL), lambda i, j: (i, 0)),
        out_shape=jax.ShapeDtypeStruct((NQ_PAD, D_MODEL), jnp.float32),
        scratch_shapes=[
            pltpu.VMEM((BQ2, D_MODEL), jnp.float32),
            pltpu.VMEM((BQ2, 128), jnp.float32),
            pltpu.VMEM((BQ2, 128), jnp.float32),
        ],
    )(qf, k_feat, v_feat, qp, kpt, kth, cut, g2, b2)
    return out[:NQ]


# speedup vs baseline: 1.0655x; 1.0655x over previous
import jax
import jax.numpy as jnp
from jax.experimental import pallas as pl
from jax.experimental.pallas import tpu as pltpu

D_MODEL = 512
K_NEAREST = 100
NQ = 3000
NKV = 16384
LN_EPS = 1e-5

BQ = 128
NQ_PAD = 3072
GRIDQ = NQ_PAD // BQ
SCALE = D_MODEL ** (-0.5)
INF_BITS = 0x7F800000

CK1 = 2048
NC1 = NKV // CK1
CK2 = 2048
NC2 = NKV // CK2
BQ2 = 512
GRIDQ2 = NQ_PAD // BQ2


def _dist_bits(qp_ref, kx, ky, kz):
    qx = qp_ref[:, 0:1]
    qy = qp_ref[:, 1:2]
    qz = qp_ref[:, 2:3]
    dx = qx - kx
    dy = qy - ky
    dz = qz - kz
    d2 = dx * dx + dy * dy + dz * dz
    return jax.lax.bitcast_convert_type(jnp.sqrt(d2), jnp.int32)


def _thresh_body(qp_ref, kpt_ref, kth_ref, cut_ref, bits_ref,
                 lo_ref, hi_ref, t_ref, fnd_ref, cl_ref, ch_ref):
    lo0 = jnp.full((BQ, 1), INF_BITS, jnp.int32)
    hi0 = jnp.zeros((BQ, 1), jnp.int32)
    for c in range(NC1):
        sl = pl.ds(c * CK1, CK1)
        b = _dist_bits(qp_ref, kpt_ref[0:1, sl],
                       kpt_ref[1:2, sl], kpt_ref[2:3, sl])
        bits_ref[:, sl] = b
        lo0 = jnp.minimum(lo0, jnp.min(b, axis=1, keepdims=True))
        hi0 = jnp.maximum(hi0, jnp.max(b, axis=1, keepdims=True))

    def count_le(mid):
        tot = jnp.zeros((BQ, 1), jnp.int32)
        for c in range(NC1):
            sl = pl.ds(c * CK1, CK1)
            tot = tot + jnp.sum((bits_ref[:, sl] <= mid).astype(jnp.int32),
                                axis=1, keepdims=True)
        return tot

    lo_ref[...] = jnp.broadcast_to(lo0 - 1, (BQ, 128))
    hi_ref[...] = jnp.broadcast_to(hi0, (BQ, 128))
    t_ref[...] = jnp.zeros((BQ, 128), jnp.int32)
    fnd_ref[...] = jnp.zeros((BQ, 128), jnp.int32)
    cl_ref[...] = jnp.zeros((BQ, 128), jnp.float32)
    ch_ref[...] = jnp.full((BQ, 128), float(NKV), jnp.float32)

    def bs_cond(carry):
        it, notdone = carry
        return jnp.logical_and(it < 40, notdone)

    def bs_step(carry):
        it, _ = carry
        lo = lo_ref[:, 0:1]
        hi = hi_ref[:, 0:1]
        found = fnd_ref[:, 0:1]
        cl = cl_ref[:, 0:1]
        ch = ch_ref[:, 0:1]

        lov = jax.lax.bitcast_convert_type(lo, jnp.float32)
        hiv = jax.lax.bitcast_convert_type(hi, jnp.float32)
        frac = (K_NEAREST - cl) / jnp.maximum(ch - cl, 1.0)
        frac = jnp.clip(frac, 0.002, 0.998)
        lo3 = lov * lov * lov
        hi3 = hiv * hiv * hiv
        t3 = jnp.maximum(lo3 + frac * (hi3 - lo3), 1e-30)
        tv = jnp.exp(jnp.log(t3) * (1.0 / 3.0))
        interp_mid = jax.lax.bitcast_convert_type(tv, jnp.int32)

        bisect_mid = lo + jax.lax.div(hi - lo, 2)
        mid = jnp.where(it < 16, interp_mid, bisect_mid)
        mid = jnp.clip(mid, lo + 1, hi - 1)

        cnt = count_le(mid)
        newly = (cnt == K_NEAREST) & (found == 0)
        t_ref[...] = jnp.broadcast_to(
            jnp.where(newly, mid, t_ref[:, 0:1]), (BQ, 128))
        found = found | newly.astype(jnp.int32)
        fnd_ref[...] = jnp.broadcast_to(found, (BQ, 128))
        take = cnt >= K_NEAREST
        lo_n = jnp.where(take, lo, mid)
        hi_n = jnp.where(take, mid, hi)
        lo_ref[...] = jnp.broadcast_to(lo_n, (BQ, 128))
        hi_ref[...] = jnp.broadcast_to(hi_n, (BQ, 128))
        cntf = cnt.astype(jnp.float32)
        cl_ref[...] = jnp.broadcast_to(jnp.where(take, cl, cntf), (BQ, 128))
        ch_ref[...] = jnp.broadcast_to(jnp.where(take, cntf, ch), (BQ, 128))
        resolved = (found == 1) | (hi_n - lo_n <= 1)
        return it + 1, jnp.logical_not(jnp.all(resolved))

    jax.lax.while_loop(bs_cond, bs_step, (jnp.int32(0), True))
    found = fnd_ref[:, 0:1]
    kth = jnp.where(found == 1, t_ref[:, 0:1], hi_ref[:, 0:1])
    tie = jnp.logical_not(jnp.all(found == 1))

    def tie_fix(_):
        def count_lt():
            tot = jnp.zeros((BQ, 1), jnp.int32)
            for c in range(NC1):
                sl = pl.ds(c * CK1, CK1)
                tot = tot + jnp.sum((bits_ref[:, sl] < kth).astype(jnp.int32),
                                    axis=1, keepdims=True)
            return tot

        need = K_NEAREST - count_lt()
        iot = jax.lax.broadcasted_iota(jnp.int32, (BQ, CK1), 1)

        def count_eq_upto(mid):
            tot = jnp.zeros((BQ, 1), jnp.int32)
            for c in range(NC1):
                sl = pl.ds(c * CK1, CK1)
                hit = (bits_ref[:, sl] == kth) & ((iot + c * CK1) <= mid)
                tot = tot + jnp.sum(hit.astype(jnp.int32), axis=1,
                                    keepdims=True)
            return tot

        def ix_step(_, lohi):
            lo, hi = lohi
            mid = lo + jax.lax.div(hi - lo, 2)
            take = count_eq_upto(mid) >= need
            return jnp.where(take, lo, mid + 1), jnp.where(take, mid, hi)

        lo = jnp.zeros((BQ, 1), jnp.int32)
        hi = jnp.full((BQ, 1), NKV - 1, jnp.int32)
        _, cut = jax.lax.fori_loop(0, 14, ix_step, (lo, hi))
        return cut

    cut = jax.lax.cond(tie, tie_fix,
                       lambda _: jnp.full((BQ, 1), NKV - 1, jnp.int32),
                       None)
    kth_ref[...] = jnp.broadcast_to(kth, (BQ, 128))
    cut_ref[...] = jnp.broadcast_to(cut, (BQ, 128))


def _attn_body(qf_ref, kf_ref, vf_ref, qp_ref, kpt_ref, kth_ref, cut_ref,
               g_ref, b_ref, out_ref, acc_ref, m_ref, l_ref):
    j = pl.program_id(1)

    bits = _dist_bits(qp_ref, kpt_ref[0:1, :], kpt_ref[1:2, :],
                      kpt_ref[2:3, :])
    kth = kth_ref[:, 0:1]
    cut = cut_ref[:, 0:1]
    iot = jax.lax.broadcasted_iota(jnp.int32, (BQ2, CK2), 1) + j * CK2
    mask = (bits < kth) | ((bits == kth) & (iot <= cut))

    s = jax.lax.dot_general(qf_ref[...], kf_ref[...],
                            (((1,), (1,)), ((), ())),
                            preferred_element_type=jnp.float32) * SCALE
    sm = jnp.where(mask, s, -jnp.inf)
    mc = jnp.max(sm, axis=1, keepdims=True)

    @pl.when(j == 0)
    def _init():
        m_ref[...] = jnp.broadcast_to(mc, (BQ2, 128))
        p = jnp.where(mask, jnp.exp(s - mc), 0.0)
        l_ref[...] = jnp.broadcast_to(
            jnp.sum(p, axis=1, keepdims=True), (BQ2, 128))
        acc_ref[...] = jax.lax.dot_general(
            p, vf_ref[...], (((1,), (0,)), ((), ())),
            preferred_element_type=jnp.float32)

    @pl.when(j > 0)
    def _update():
        m_old = m_ref[:, 0:1]
        m_new = jnp.maximum(m_old, mc)
        corr = jnp.where(m_old == -jnp.inf, 0.0, jnp.exp(m_old - m_new))
        p = jnp.where(mask, jnp.exp(s - m_new), 0.0)
        m_ref[...] = jnp.broadcast_to(m_new, (BQ2, 128))
        l_ref[...] = (l_ref[...] * corr +
                      jnp.broadcast_to(jnp.sum(p, axis=1, keepdims=True),
                                       (BQ2, 128)))
        acc_ref[...] = (acc_ref[...] * corr +
                        jax.lax.dot_general(
                            p, vf_ref[...], (((1,), (0,)), ((), ())),
                            preferred_element_type=jnp.float32))

    @pl.when(j == NC2 - 1)
    def _finalize():
        x = acc_ref[...] / l_ref[:, 0:1]
        t = x + x
        mu = jnp.mean(t, axis=1, keepdims=True)
        var = jnp.mean((t - mu) ** 2, axis=1, keepdims=True)
        xh = (t - mu) / jnp.sqrt(var + LN_EPS)
        out_ref[...] = xh * g_ref[0:1, :] + b_ref[0:1, :]


@jax.jit
def kernel(res_feat, q_feat, k_feat, v_feat, q_pos, k_pos, ln_gamma, ln_beta):
    del res_feat
    qf = jnp.pad(q_feat, ((0, NQ_PAD - NQ), (0, 0)))
    qp = jnp.pad(q_pos, ((0, NQ_PAD - NQ), (0, 125)))
    kpt = jnp.pad(k_pos.T, ((0, 5), (0, 0)))
    g2 = jnp.broadcast_to(ln_gamma[None, :], (8, D_MODEL))
    b2 = jnp.broadcast_to(ln_beta[None, :], (8, D_MODEL))

    kth, cut = pl.pallas_call(
        _thresh_body,
        grid=(GRIDQ,),
        in_specs=[
            pl.BlockSpec((BQ, 128), lambda i: (i, 0)),
            pl.BlockSpec((8, NKV), lambda i: (0, 0)),
        ],
        out_specs=[
            pl.BlockSpec((BQ, 128), lambda i: (i, 0)),
            pl.BlockSpec((BQ, 128), lambda i: (i, 0)),
        ],
        out_shape=[
            jax.ShapeDtypeStruct((NQ_PAD, 128), jnp.int32),
            jax.ShapeDtypeStruct((NQ_PAD, 128), jnp.int32),
        ],
        scratch_shapes=[pltpu.VMEM((BQ, NKV), jnp.int32),
                        pltpu.VMEM((BQ, 128), jnp.int32),
                        pltpu.VMEM((BQ, 128), jnp.int32),
                        pltpu.VMEM((BQ, 128), jnp.int32),
                        pltpu.VMEM((BQ, 128), jnp.int32),
                        pltpu.VMEM((BQ, 128), jnp.float32),
                        pltpu.VMEM((BQ, 128), jnp.float32)],
    )(qp, kpt)

    out = pl.pallas_call(
        _attn_body,
        grid=(GRIDQ2, NC2),
        in_specs=[
            pl.BlockSpec((BQ2, D_MODEL), lambda i, j: (i, 0)),
            pl.BlockSpec((CK2, D_MODEL), lambda i, j: (j, 0)),
            pl.BlockSpec((CK2, D_MODEL), lambda i, j: (j, 0)),
            pl.BlockSpec((BQ2, 128), lambda i, j: (i, 0)),
            pl.BlockSpec((8, CK2), lambda i, j: (0, j)),
            pl.BlockSpec((BQ2, 128), lambda i, j: (i, 0)),
            pl.BlockSpec((BQ2, 128), lambda i, j: (i, 0)),
            pl.BlockSpec((8, D_MODEL), lambda i, j: (0, 0)),
            pl.BlockSpec((8, D_MODEL), lambda i, j: (0, 0)),
        ],
        out_specs=pl.BlockSpec((BQ2, D_MODEL), lambda i, j: (i, 0)),
        out_shape=jax.ShapeDtypeStruct((NQ_PAD, D_MODEL), jnp.float32),
        scratch_shapes=[
            pltpu.VMEM((BQ2, D_MODEL), jnp.float32),
            pltpu.VMEM((BQ2, 128), jnp.float32),
            pltpu.VMEM((BQ2, 128), jnp.float32),
        ],
    )(qf, k_feat, v_feat, qp, kpt, kth, cut, g2, b2)
    return out[:NQ]


# speedup vs baseline: 1.1747x; 1.1025x over previous
import jax
import jax.numpy as jnp
from jax.experimental import pallas as pl
from jax.experimental.pallas import tpu as pltpu

D_MODEL = 512
K_NEAREST = 100
NQ = 3000
NKV = 16384
LN_EPS = 1e-5

BQ = 128
NQ_PAD = 3072
GRIDQ = NQ_PAD // BQ
SCALE = D_MODEL ** (-0.5)
INF_BITS = 0x7F800000

CK1 = 2048
NC1 = NKV // CK1
CK2 = 2048
NC2 = NKV // CK2
BQ2 = 512
GRIDQ2 = NQ_PAD // BQ2


def _dist_bits(qp_ref, kx, ky, kz):
    qx = qp_ref[:, 0:1]
    qy = qp_ref[:, 1:2]
    qz = qp_ref[:, 2:3]
    dx = qx - kx
    dy = qy - ky
    dz = qz - kz
    d2 = dx * dx + dy * dy + dz * dz
    return jax.lax.bitcast_convert_type(d2, jnp.int32)


def _thresh_body(qp_ref, kpt_ref, kth_ref, cut_ref, bits_ref,
                 lo_ref, hi_ref, t_ref, fnd_ref, cl_ref, ch_ref):
    lo0 = jnp.full((BQ, 1), INF_BITS, jnp.int32)
    hi0 = jnp.zeros((BQ, 1), jnp.int32)
    for c in range(NC1):
        sl = pl.ds(c * CK1, CK1)
        b = _dist_bits(qp_ref, kpt_ref[0:1, sl],
                       kpt_ref[1:2, sl], kpt_ref[2:3, sl])
        bits_ref[:, sl] = b
        lo0 = jnp.minimum(lo0, jnp.min(b, axis=1, keepdims=True))
        hi0 = jnp.maximum(hi0, jnp.max(b, axis=1, keepdims=True))

    def count_le(mid):
        tot = jnp.zeros((BQ, 1), jnp.int32)
        for c in range(NC1):
            sl = pl.ds(c * CK1, CK1)
            tot = tot + jnp.sum((bits_ref[:, sl] <= mid).astype(jnp.int32),
                                axis=1, keepdims=True)
        return tot

    lo_ref[...] = jnp.broadcast_to(lo0 - 1, (BQ, 128))
    hi_ref[...] = jnp.broadcast_to(hi0, (BQ, 128))
    t_ref[...] = jnp.zeros((BQ, 128), jnp.int32)
    fnd_ref[...] = jnp.zeros((BQ, 128), jnp.int32)
    cl_ref[...] = jnp.zeros((BQ, 128), jnp.float32)
    ch_ref[...] = jnp.full((BQ, 128), float(NKV), jnp.float32)

    def bs_cond(carry):
        it, notdone = carry
        return jnp.logical_and(it < 40, notdone)

    def bs_step(carry):
        it, _ = carry
        lo = lo_ref[:, 0:1]
        hi = hi_ref[:, 0:1]
        found = fnd_ref[:, 0:1]
        cl = cl_ref[:, 0:1]
        ch = ch_ref[:, 0:1]

        lov = jax.lax.bitcast_convert_type(lo, jnp.float32)
        hiv = jax.lax.bitcast_convert_type(hi, jnp.float32)
        frac = (K_NEAREST - cl) / jnp.maximum(ch - cl, 1.0)
        frac = jnp.clip(frac, 0.002, 0.998)
        lo15 = lov * jnp.sqrt(jnp.maximum(lov, 0.0))
        hi15 = hiv * jnp.sqrt(jnp.maximum(hiv, 0.0))
        t15 = jnp.maximum(lo15 + frac * (hi15 - lo15), 1e-30)
        tv = jnp.exp(jnp.log(t15) * (2.0 / 3.0))
        interp_mid = jax.lax.bitcast_convert_type(tv, jnp.int32)

        bisect_mid = lo + jax.lax.div(hi - lo, 2)
        mid = jnp.where(it < 16, interp_mid, bisect_mid)
        mid = jnp.clip(mid, lo + 1, hi - 1)

        cnt = count_le(mid)
        newly = (cnt == K_NEAREST) & (found == 0)
        t_ref[...] = jnp.broadcast_to(
            jnp.where(newly, mid, t_ref[:, 0:1]), (BQ, 128))
        found = found | newly.astype(jnp.int32)
        fnd_ref[...] = jnp.broadcast_to(found, (BQ, 128))
        take = cnt >= K_NEAREST
        lo_n = jnp.where(take, lo, mid)
        hi_n = jnp.where(take, mid, hi)
        lo_ref[...] = jnp.broadcast_to(lo_n, (BQ, 128))
        hi_ref[...] = jnp.broadcast_to(hi_n, (BQ, 128))
        cntf = cnt.astype(jnp.float32)
        cl_ref[...] = jnp.broadcast_to(jnp.where(take, cl, cntf), (BQ, 128))
        ch_ref[...] = jnp.broadcast_to(jnp.where(take, cntf, ch), (BQ, 128))
        resolved = (found == 1) | (hi_n - lo_n <= 1)
        return it + 1, jnp.logical_not(jnp.all(resolved))

    jax.lax.while_loop(bs_cond, bs_step, (jnp.int32(0), True))
    found = fnd_ref[:, 0:1]
    kth = jnp.where(found == 1, t_ref[:, 0:1], hi_ref[:, 0:1])
    tie = jnp.logical_not(jnp.all(found == 1))

    def tie_fix(_):
        def count_lt():
            tot = jnp.zeros((BQ, 1), jnp.int32)
            for c in range(NC1):
                sl = pl.ds(c * CK1, CK1)
                tot = tot + jnp.sum((bits_ref[:, sl] < kth).astype(jnp.int32),
                                    axis=1, keepdims=True)
            return tot

        need = K_NEAREST - count_lt()
        iot = jax.lax.broadcasted_iota(jnp.int32, (BQ, CK1), 1)

        def count_eq_upto(mid):
            tot = jnp.zeros((BQ, 1), jnp.int32)
            for c in range(NC1):
                sl = pl.ds(c * CK1, CK1)
                hit = (bits_ref[:, sl] == kth) & ((iot + c * CK1) <= mid)
                tot = tot + jnp.sum(hit.astype(jnp.int32), axis=1,
                                    keepdims=True)
            return tot

        def ix_step(_, lohi):
            lo, hi = lohi
            mid = lo + jax.lax.div(hi - lo, 2)
            take = count_eq_upto(mid) >= need
            return jnp.where(take, lo, mid + 1), jnp.where(take, mid, hi)

        lo = jnp.zeros((BQ, 1), jnp.int32)
        hi = jnp.full((BQ, 1), NKV - 1, jnp.int32)
        _, cut = jax.lax.fori_loop(0, 14, ix_step, (lo, hi))
        return cut

    cut = jax.lax.cond(tie, tie_fix,
                       lambda _: jnp.full((BQ, 1), NKV - 1, jnp.int32),
                       None)
    kth_ref[...] = jnp.broadcast_to(kth, (BQ, 128))
    cut_ref[...] = jnp.broadcast_to(cut, (BQ, 128))


def _attn_body(qf_ref, kf_ref, vf_ref, qp_ref, kpt_ref, kth_ref, cut_ref,
               g_ref, b_ref, out_ref, acc_ref, m_ref, l_ref):
    j = pl.program_id(1)

    bits = _dist_bits(qp_ref, kpt_ref[0:1, :], kpt_ref[1:2, :],
                      kpt_ref[2:3, :])
    kth = kth_ref[:, 0:1]
    cut = cut_ref[:, 0:1]
    iot = jax.lax.broadcasted_iota(jnp.int32, (BQ2, CK2), 1) + j * CK2
    mask = (bits < kth) | ((bits == kth) & (iot <= cut))

    s = jax.lax.dot_general(qf_ref[...], kf_ref[...],
                            (((1,), (1,)), ((), ())),
                            preferred_element_type=jnp.float32) * SCALE
    sm = jnp.where(mask, s, -jnp.inf)
    mc = jnp.max(sm, axis=1, keepdims=True)

    @pl.when(j == 0)
    def _init():
        m_ref[...] = jnp.broadcast_to(mc, (BQ2, 128))
        p = jnp.where(mask, jnp.exp(s - mc), 0.0)
        l_ref[...] = jnp.broadcast_to(
            jnp.sum(p, axis=1, keepdims=True), (BQ2, 128))
        acc_ref[...] = jax.lax.dot_general(
            p, vf_ref[...], (((1,), (0,)), ((), ())),
            preferred_element_type=jnp.float32)

    @pl.when(j > 0)
    def _update():
        m_old = m_ref[:, 0:1]
        m_new = jnp.maximum(m_old, mc)
        corr = jnp.where(m_old == -jnp.inf, 0.0, jnp.exp(m_old - m_new))
        p = jnp.where(mask, jnp.exp(s - m_new), 0.0)
        m_ref[...] = jnp.broadcast_to(m_new, (BQ2, 128))
        l_ref[...] = (l_ref[...] * corr +
                      jnp.broadcast_to(jnp.sum(p, axis=1, keepdims=True),
                                       (BQ2, 128)))
        acc_ref[...] = (acc_ref[...] * corr +
                        jax.lax.dot_general(
                            p, vf_ref[...], (((1,), (0,)), ((), ())),
                            preferred_element_type=jnp.float32))

    @pl.when(j == NC2 - 1)
    def _finalize():
        x = acc_ref[...] / l_ref[:, 0:1]
        t = x + x
        mu = jnp.mean(t, axis=1, keepdims=True)
        var = jnp.mean((t - mu) ** 2, axis=1, keepdims=True)
        xh = (t - mu) / jnp.sqrt(var + LN_EPS)
        out_ref[...] = xh * g_ref[0:1, :] + b_ref[0:1, :]


@jax.jit
def kernel(res_feat, q_feat, k_feat, v_feat, q_pos, k_pos, ln_gamma, ln_beta):
    del res_feat
    qf = jnp.pad(q_feat, ((0, NQ_PAD - NQ), (0, 0)))
    qp = jnp.pad(q_pos, ((0, NQ_PAD - NQ), (0, 125)))
    kpt = jnp.pad(k_pos.T, ((0, 5), (0, 0)))
    g2 = jnp.broadcast_to(ln_gamma[None, :], (8, D_MODEL))
    b2 = jnp.broadcast_to(ln_beta[None, :], (8, D_MODEL))

    kth, cut = pl.pallas_call(
        _thresh_body,
        grid=(GRIDQ,),
        in_specs=[
            pl.BlockSpec((BQ, 128), lambda i: (i, 0)),
            pl.BlockSpec((8, NKV), lambda i: (0, 0)),
        ],
        out_specs=[
            pl.BlockSpec((BQ, 128), lambda i: (i, 0)),
            pl.BlockSpec((BQ, 128), lambda i: (i, 0)),
        ],
        out_shape=[
            jax.ShapeDtypeStruct((NQ_PAD, 128), jnp.int32),
            jax.ShapeDtypeStruct((NQ_PAD, 128), jnp.int32),
        ],
        scratch_shapes=[pltpu.VMEM((BQ, NKV), jnp.int32),
                        pltpu.VMEM((BQ, 128), jnp.int32),
                        pltpu.VMEM((BQ, 128), jnp.int32),
                        pltpu.VMEM((BQ, 128), jnp.int32),
                        pltpu.VMEM((BQ, 128), jnp.int32),
                        pltpu.VMEM((BQ, 128), jnp.float32),
                        pltpu.VMEM((BQ, 128), jnp.float32)],
    )(qp, kpt)

    out = pl.pallas_call(
        _attn_body,
        grid=(GRIDQ2, NC2),
        in_specs=[
            pl.BlockSpec((BQ2, D_MODEL), lambda i, j: (i, 0)),
            pl.BlockSpec((CK2, D_MODEL), lambda i, j: (j, 0)),
            pl.BlockSpec((CK2, D_MODEL), lambda i, j: (j, 0)),
            pl.BlockSpec((BQ2, 128), lambda i, j: (i, 0)),
            pl.BlockSpec((8, CK2), lambda i, j: (0, j)),
            pl.BlockSpec((BQ2, 128), lambda i, j: (i, 0)),
            pl.BlockSpec((BQ2, 128), lambda i, j: (i, 0)),
            pl.BlockSpec((8, D_MODEL), lambda i, j: (0, 0)),
            pl.BlockSpec((8, D_MODEL), lambda i, j: (0, 0)),
        ],
        out_specs=pl.BlockSpec((BQ2, D_MODEL), lambda i, j: (i, 0)),
        out_shape=jax.ShapeDtypeStruct((NQ_PAD, D_MODEL), jnp.float32),
        scratch_shapes=[
            pltpu.VMEM((BQ2, D_MODEL), jnp.float32),
            pltpu.VMEM((BQ2, 128), jnp.float32),
            pltpu.VMEM((BQ2, 128), jnp.float32),
        ],
    )(qf, k_feat, v_feat, qp, kpt, kth, cut, g2, b2)
    return out[:NQ]


# speedup vs baseline: 1.2066x; 1.0271x over previous
import jax
import jax.numpy as jnp
from jax.experimental import pallas as pl
from jax.experimental.pallas import tpu as pltpu

D_MODEL = 512
K_NEAREST = 100
NQ = 3000
NKV = 16384
LN_EPS = 1e-5

BQ = 128
NQ_PAD = 3072
GRIDQ = NQ_PAD // BQ
SCALE = D_MODEL ** (-0.5)
INF_BITS = 0x7F800000

CK1 = 4096
NC1 = NKV // CK1
CK2 = 2048
NC2 = NKV // CK2
BQ2 = 512
GRIDQ2 = NQ_PAD // BQ2


def _dist_bits(qp_ref, kx, ky, kz):
    qx = qp_ref[:, 0:1]
    qy = qp_ref[:, 1:2]
    qz = qp_ref[:, 2:3]
    dx = qx - kx
    dy = qy - ky
    dz = qz - kz
    d2 = dx * dx + dy * dy + dz * dz
    return jax.lax.bitcast_convert_type(d2, jnp.int32)


def _thresh_body(qp_ref, kpt_ref, kth_ref, cut_ref, bits_ref,
                 lo_ref, hi_ref, t_ref, fnd_ref, cl_ref, ch_ref):
    lo0 = jnp.full((BQ, 1), INF_BITS, jnp.int32)
    hi0 = jnp.zeros((BQ, 1), jnp.int32)
    for c in range(NC1):
        sl = pl.ds(c * CK1, CK1)
        b = _dist_bits(qp_ref, kpt_ref[0:1, sl],
                       kpt_ref[1:2, sl], kpt_ref[2:3, sl])
        bits_ref[:, sl] = b
        lo0 = jnp.minimum(lo0, jnp.min(b, axis=1, keepdims=True))
        hi0 = jnp.maximum(hi0, jnp.max(b, axis=1, keepdims=True))

    def count_le(mid):
        tot = jnp.zeros((BQ, 1), jnp.int32)
        for c in range(NC1):
            sl = pl.ds(c * CK1, CK1)
            tot = tot + jnp.sum((bits_ref[:, sl] <= mid).astype(jnp.int32),
                                axis=1, keepdims=True)
        return tot

    lo_ref[...] = jnp.broadcast_to(lo0 - 1, (BQ, 128))
    hi_ref[...] = jnp.broadcast_to(hi0, (BQ, 128))
    t_ref[...] = jnp.zeros((BQ, 128), jnp.int32)
    fnd_ref[...] = jnp.zeros((BQ, 128), jnp.int32)
    cl_ref[...] = jnp.zeros((BQ, 128), jnp.float32)
    ch_ref[...] = jnp.full((BQ, 128), float(NKV), jnp.float32)

    def bs_cond(carry):
        it, notdone = carry
        return jnp.logical_and(it < 40, notdone)

    def bs_step(carry):
        it, _ = carry
        lo = lo_ref[:, 0:1]
        hi = hi_ref[:, 0:1]
        found = fnd_ref[:, 0:1]
        cl = cl_ref[:, 0:1]
        ch = ch_ref[:, 0:1]

        lov = jax.lax.bitcast_convert_type(lo, jnp.float32)
        hiv = jax.lax.bitcast_convert_type(hi, jnp.float32)
        frac = (K_NEAREST - cl) / jnp.maximum(ch - cl, 1.0)
        frac = jnp.clip(frac, 0.002, 0.998)
        lo15 = lov * jnp.sqrt(jnp.maximum(lov, 0.0))
        hi15 = hiv * jnp.sqrt(jnp.maximum(hiv, 0.0))
        t15 = jnp.maximum(lo15 + frac * (hi15 - lo15), 1e-30)
        tv = jnp.exp(jnp.log(t15) * (2.0 / 3.0))
        interp_mid = jax.lax.bitcast_convert_type(tv, jnp.int32)

        bisect_mid = lo + jax.lax.div(hi - lo, 2)
        mid = jnp.where(it < 16, interp_mid, bisect_mid)
        mid = jnp.clip(mid, lo + 1, hi - 1)

        cnt = count_le(mid)
        newly = (cnt == K_NEAREST) & (found == 0)
        t_ref[...] = jnp.broadcast_to(
            jnp.where(newly, mid, t_ref[:, 0:1]), (BQ, 128))
        found = found | newly.astype(jnp.int32)
        fnd_ref[...] = jnp.broadcast_to(found, (BQ, 128))
        take = cnt >= K_NEAREST
        lo_n = jnp.where(take, lo, mid)
        hi_n = jnp.where(take, mid, hi)
        lo_ref[...] = jnp.broadcast_to(lo_n, (BQ, 128))
        hi_ref[...] = jnp.broadcast_to(hi_n, (BQ, 128))
        cntf = cnt.astype(jnp.float32)
        cl_ref[...] = jnp.broadcast_to(jnp.where(take, cl, cntf), (BQ, 128))
        ch_ref[...] = jnp.broadcast_to(jnp.where(take, cntf, ch), (BQ, 128))
        resolved = (found == 1) | (hi_n - lo_n <= 1)
        return it + 1, jnp.logical_not(jnp.all(resolved))

    jax.lax.while_loop(bs_cond, bs_step, (jnp.int32(0), True))
    found = fnd_ref[:, 0:1]
    kth = jnp.where(found == 1, t_ref[:, 0:1], hi_ref[:, 0:1])
    tie = jnp.logical_not(jnp.all(found == 1))

    def tie_fix(_):
        def count_lt():
            tot = jnp.zeros((BQ, 1), jnp.int32)
            for c in range(NC1):
                sl = pl.ds(c * CK1, CK1)
                tot = tot + jnp.sum((bits_ref[:, sl] < kth).astype(jnp.int32),
                                    axis=1, keepdims=True)
            return tot

        need = K_NEAREST - count_lt()
        iot = jax.lax.broadcasted_iota(jnp.int32, (BQ, CK1), 1)

        def count_eq_upto(mid):
            tot = jnp.zeros((BQ, 1), jnp.int32)
            for c in range(NC1):
                sl = pl.ds(c * CK1, CK1)
                hit = (bits_ref[:, sl] == kth) & ((iot + c * CK1) <= mid)
                tot = tot + jnp.sum(hit.astype(jnp.int32), axis=1,
                                    keepdims=True)
            return tot

        def ix_step(_, lohi):
            lo, hi = lohi
            mid = lo + jax.lax.div(hi - lo, 2)
            take = count_eq_upto(mid) >= need
            return jnp.where(take, lo, mid + 1), jnp.where(take, mid, hi)

        lo = jnp.zeros((BQ, 1), jnp.int32)
        hi = jnp.full((BQ, 1), NKV - 1, jnp.int32)
        _, cut = jax.lax.fori_loop(0, 14, ix_step, (lo, hi))
        return cut

    cut = jax.lax.cond(tie, tie_fix,
                       lambda _: jnp.full((BQ, 1), NKV - 1, jnp.int32),
                       None)
    kth_ref[...] = jnp.broadcast_to(kth, (BQ, 128))
    cut_ref[...] = jnp.broadcast_to(cut, (BQ, 128))


def _attn_body(qf_ref, kf_ref, vf_ref, qp_ref, kpt_ref, kth_ref, cut_ref,
               g_ref, b_ref, out_ref, acc_ref, m_ref, l_ref):
    j = pl.program_id(1)

    bits = _dist_bits(qp_ref, kpt_ref[0:1, :], kpt_ref[1:2, :],
                      kpt_ref[2:3, :])
    kth = kth_ref[:, 0:1]
    cut = cut_ref[:, 0:1]
    iot = jax.lax.broadcasted_iota(jnp.int32, (BQ2, CK2), 1) + j * CK2
    mask = (bits < kth) | ((bits == kth) & (iot <= cut))

    s = jax.lax.dot_general(qf_ref[...], kf_ref[...],
                            (((1,), (1,)), ((), ())),
                            preferred_element_type=jnp.float32) * SCALE
    sm = jnp.where(mask, s, -jnp.inf)
    mc = jnp.max(sm, axis=1, keepdims=True)

    @pl.when(j == 0)
    def _init():
        m_ref[...] = jnp.broadcast_to(mc, (BQ2, 128))
        p = jnp.where(mask, jnp.exp(s - mc), 0.0)
        l_ref[...] = jnp.broadcast_to(
            jnp.sum(p, axis=1, keepdims=True), (BQ2, 128))
        acc_ref[...] = jax.lax.dot_general(
            p, vf_ref[...], (((1,), (0,)), ((), ())),
            preferred_element_type=jnp.float32)

    @pl.when(j > 0)
    def _update():
        m_old = m_ref[:, 0:1]
        m_new = jnp.maximum(m_old, mc)
        corr = jnp.where(m_old == -jnp.inf, 0.0, jnp.exp(m_old - m_new))
        p = jnp.where(mask, jnp.exp(s - m_new), 0.0)
        m_ref[...] = jnp.broadcast_to(m_new, (BQ2, 128))
        l_ref[...] = (l_ref[...] * corr +
                      jnp.broadcast_to(jnp.sum(p, axis=1, keepdims=True),
                                       (BQ2, 128)))
        acc_ref[...] = (acc_ref[...] * corr +
                        jax.lax.dot_general(
                            p, vf_ref[...], (((1,), (0,)), ((), ())),
                            preferred_element_type=jnp.float32))

    @pl.when(j == NC2 - 1)
    def _finalize():
        x = acc_ref[...] / l_ref[:, 0:1]
        t = x + x
        mu = jnp.mean(t, axis=1, keepdims=True)
        var = jnp.mean((t - mu) ** 2, axis=1, keepdims=True)
        xh = (t - mu) / jnp.sqrt(var + LN_EPS)
        out_ref[...] = xh * g_ref[0:1, :] + b_ref[0:1, :]


@jax.jit
def kernel(res_feat, q_feat, k_feat, v_feat, q_pos, k_pos, ln_gamma, ln_beta):
    del res_feat
    qf = jnp.pad(q_feat, ((0, NQ_PAD - NQ), (0, 0)))
    qp = jnp.pad(q_pos, ((0, NQ_PAD - NQ), (0, 125)))
    kpt = jnp.pad(k_pos.T, ((0, 5), (0, 0)))
    g2 = jnp.broadcast_to(ln_gamma[None, :], (8, D_MODEL))
    b2 = jnp.broadcast_to(ln_beta[None, :], (8, D_MODEL))

    kth, cut = pl.pallas_call(
        _thresh_body,
        grid=(GRIDQ,),
        in_specs=[
            pl.BlockSpec((BQ, 128), lambda i: (i, 0)),
            pl.BlockSpec((8, NKV), lambda i: (0, 0)),
        ],
        out_specs=[
            pl.BlockSpec((BQ, 128), lambda i: (i, 0)),
            pl.BlockSpec((BQ, 128), lambda i: (i, 0)),
        ],
        out_shape=[
            jax.ShapeDtypeStruct((NQ_PAD, 128), jnp.int32),
            jax.ShapeDtypeStruct((NQ_PAD, 128), jnp.int32),
        ],
        scratch_shapes=[pltpu.VMEM((BQ, NKV), jnp.int32),
                        pltpu.VMEM((BQ, 128), jnp.int32),
                        pltpu.VMEM((BQ, 128), jnp.int32),
                        pltpu.VMEM((BQ, 128), jnp.int32),
                        pltpu.VMEM((BQ, 128), jnp.int32),
                        pltpu.VMEM((BQ, 128), jnp.float32),
                        pltpu.VMEM((BQ, 128), jnp.float32)],
    )(qp, kpt)

    out = pl.pallas_call(
        _attn_body,
        grid=(GRIDQ2, NC2),
        in_specs=[
            pl.BlockSpec((BQ2, D_MODEL), lambda i, j: (i, 0)),
            pl.BlockSpec((CK2, D_MODEL), lambda i, j: (j, 0)),
            pl.BlockSpec((CK2, D_MODEL), lambda i, j: (j, 0)),
            pl.BlockSpec((BQ2, 128), lambda i, j: (i, 0)),
            pl.BlockSpec((8, CK2), lambda i, j: (0, j)),
            pl.BlockSpec((BQ2, 128), lambda i, j: (i, 0)),
            pl.BlockSpec((BQ2, 128), lambda i, j: (i, 0)),
            pl.BlockSpec((8, D_MODEL), lambda i, j: (0, 0)),
            pl.BlockSpec((8, D_MODEL), lambda i, j: (0, 0)),
        ],
        out_specs=pl.BlockSpec((BQ2, D_MODEL), lambda i, j: (i, 0)),
        out_shape=jax.ShapeDtypeStruct((NQ_PAD, D_MODEL), jnp.float32),
        scratch_shapes=[
            pltpu.VMEM((BQ2, D_MODEL), jnp.float32),
            pltpu.VMEM((BQ2, 128), jnp.float32),
            pltpu.VMEM((BQ2, 128), jnp.float32),
        ],
    )(qf, k_feat, v_feat, qp, kpt, kth, cut, g2, b2)
    return out[:NQ]
